# Initial kernel scaffold; baseline (speedup 1.0000x reference)
import jax
import jax.numpy as jnp
from jax.experimental import pallas as pl


def kernel(x, centroids):
    raise NotImplementedError("write your pallas kernel here")


# speedup vs baseline: 22.3313x; 22.3313x over previous
import jax
import jax.numpy as jnp
from jax.experimental import pallas as pl

_BN = 128


def _vq_argmin_kernel(x_ref, ct_ref, out_ref):
    ct = ct_ref[...]
    k = ct.shape[1]
    cnorm = jnp.sum(ct * ct, axis=0, keepdims=True)
    g = jnp.dot(x_ref[...], ct, preferred_element_type=jnp.float32,
                precision=jax.lax.Precision.HIGHEST)
    score = cnorm - 2.0 * g
    m = jnp.min(score, axis=1, keepdims=True)
    col = jax.lax.broadcasted_iota(jnp.int32, score.shape, 1)
    idx = jnp.min(jnp.where(score == m, col, k), axis=1)
    out_ref[...] = idx


def kernel(x, centroids):
    n, d = x.shape
    k = centroids.shape[0]
    ct = centroids.T
    return pl.pallas_call(
        _vq_argmin_kernel,
        grid=(n // _BN,),
        in_specs=[
            pl.BlockSpec((_BN, d), lambda i: (i, 0)),
            pl.BlockSpec((d, k), lambda i: (0, 0)),
        ],
        out_specs=pl.BlockSpec((_BN,), lambda i: (i,)),
        out_shape=jax.ShapeDtypeStruct((n,), jnp.int32),
    )(x, ct)


# speedup vs baseline: 28.0591x; 1.2565x over previous
import jax
import jax.numpy as jnp
from jax.experimental import pallas as pl
from jax.experimental.pallas import tpu as pltpu

_BN = 256


def _vq_argmin_kernel(x_ref, ct_ref, out_ref):
    ct = ct_ref[...]
    k = ct.shape[1]
    cnorm = jnp.sum(ct * ct, axis=0, keepdims=True)
    g = jnp.dot(x_ref[...], ct, preferred_element_type=jnp.float32,
                precision=jax.lax.Precision.HIGHEST)
    score = cnorm - 2.0 * g
    m = jnp.min(score, axis=1, keepdims=True)
    col = jax.lax.broadcasted_iota(jnp.int32, score.shape, 1)
    idx = jnp.min(jnp.where(score == m, col, k), axis=1)
    out_ref[...] = idx


def kernel(x, centroids):
    n, d = x.shape
    k = centroids.shape[0]
    ct = centroids.T
    return pl.pallas_call(
        _vq_argmin_kernel,
        grid=(n // _BN,),
        in_specs=[
            pl.BlockSpec((_BN, d), lambda i: (i, 0)),
            pl.BlockSpec((d, k), lambda i: (0, 0)),
        ],
        out_specs=pl.BlockSpec((_BN,), lambda i: (i,)),
        out_shape=jax.ShapeDtypeStruct((n,), jnp.int32),
        compiler_params=pltpu.CompilerParams(
            dimension_semantics=("parallel",)),
    )(x, ct)


# speedup vs baseline: 28.8021x; 1.0265x over previous
import jax
import jax.numpy as jnp
from jax.experimental import pallas as pl
from jax.experimental.pallas import tpu as pltpu

_BN = 512


def _vq_argmin_kernel(x_ref, ct_ref, out_ref):
    ct = ct_ref[...]
    k = ct.shape[1]
    cnorm = jnp.sum(ct * ct, axis=0, keepdims=True)
    g = jnp.dot(x_ref[...], ct, preferred_element_type=jnp.float32,
                precision=jax.lax.Precision.HIGHEST)
    score = cnorm - 2.0 * g
    m = jnp.min(score, axis=1, keepdims=True)
    col = jax.lax.broadcasted_iota(jnp.int32, score.shape, 1)
    idx = jnp.min(jnp.where(score == m, col, k), axis=1)
    out_ref[...] = idx


def kernel(x, centroids):
    n, d = x.shape
    k = centroids.shape[0]
    ct = centroids.T
    return pl.pallas_call(
        _vq_argmin_kernel,
        grid=(n // _BN,),
        in_specs=[
            pl.BlockSpec((_BN, d), lambda i: (i, 0)),
            pl.BlockSpec((d, k), lambda i: (0, 0)),
        ],
        out_specs=pl.BlockSpec((_BN,), lambda i: (i,)),
        out_shape=jax.ShapeDtypeStruct((n,), jnp.int32),
        compiler_params=pltpu.CompilerParams(
            dimension_semantics=("parallel",)),
    )(x, ct)


# speedup vs baseline: 36.0779x; 1.2526x over previous
import jax
import jax.numpy as jnp
from jax.experimental import pallas as pl
from jax.experimental.pallas import tpu as pltpu

_BN = 512


def _vq_argmin_kernel(x_ref, c_ref, out_ref, ct_ref):
    i = pl.program_id(0)

    @pl.when(i == 0)
    def _():
        ct_ref[...] = c_ref[...].T

    ct = ct_ref[...]
    k = ct.shape[1]
    cnorm = jnp.sum(ct * ct, axis=0, keepdims=True)
    g = jnp.dot(x_ref[...], ct, preferred_element_type=jnp.float32,
                precision=jax.lax.Precision.HIGHEST)
    score = cnorm - 2.0 * g
    m = jnp.min(score, axis=1, keepdims=True)
    col = jax.lax.broadcasted_iota(jnp.int32, score.shape, 1)
    idx = jnp.min(jnp.where(score == m, col, k), axis=1)
    out_ref[...] = idx


def kernel(x, centroids):
    n, d = x.shape
    k = centroids.shape[0]
    return pl.pallas_call(
        _vq_argmin_kernel,
        grid=(n // _BN,),
        in_specs=[
            pl.BlockSpec((_BN, d), lambda i: (i, 0)),
            pl.BlockSpec((k, d), lambda i: (0, 0)),
        ],
        out_specs=pl.BlockSpec((_BN,), lambda i: (i,)),
        out_shape=jax.ShapeDtypeStruct((n,), jnp.int32),
        scratch_shapes=[pltpu.VMEM((d, k), jnp.float32)],
        compiler_params=pltpu.CompilerParams(
            dimension_semantics=("arbitrary",)),
    )(x, centroids)


# speedup vs baseline: 36.8779x; 1.0222x over previous
import jax
import jax.numpy as jnp
from jax.experimental import pallas as pl
from jax.experimental.pallas import tpu as pltpu

_BN = 512


def _vq_argmin_kernel(x_ref, c_ref, out_ref, ct2_ref, cn_ref):
    i = pl.program_id(0)

    @pl.when(i == 0)
    def _():
        ct2 = c_ref[...].T * -2.0
        ct2_ref[...] = ct2
        cn_ref[...] = 0.25 * jnp.sum(ct2 * ct2, axis=0, keepdims=True)

    ct2 = ct2_ref[...]
    k = ct2.shape[1]
    g2 = jnp.dot(x_ref[...], ct2, preferred_element_type=jnp.float32,
                 precision=jax.lax.Precision.HIGHEST)
    score = cn_ref[...] + g2
    m = jnp.min(score, axis=1, keepdims=True)
    col = jax.lax.broadcasted_iota(jnp.int32, score.shape, 1)
    idx = jnp.min(jnp.where(score == m, col, k), axis=1)
    out_ref[...] = idx


def kernel(x, centroids):
    n, d = x.shape
    k = centroids.shape[0]
    return pl.pallas_call(
        _vq_argmin_kernel,
        grid=(n // _BN,),
        in_specs=[
            pl.BlockSpec((_BN, d), lambda i: (i, 0)),
            pl.BlockSpec((k, d), lambda i: (0, 0)),
        ],
        out_specs=pl.BlockSpec((_BN,), lambda i: (i,)),
        out_shape=jax.ShapeDtypeStruct((n,), jnp.int32),
        scratch_shapes=[pltpu.VMEM((d, k), jnp.float32),
                        pltpu.VMEM((1, k), jnp.float32)],
        compiler_params=pltpu.CompilerParams(
            dimension_semantics=("arbitrary",)),
    )(x, centroids)


# speedup vs baseline: 38.7158x; 1.0498x over previous
import jax
import jax.numpy as jnp
from jax.experimental import pallas as pl


def _vq_argmin_kernel(x_ref, c_ref, out_ref):
    ct2 = c_ref[...].T * -2.0
    k = ct2.shape[1]
    cnorm = 0.25 * jnp.sum(ct2 * ct2, axis=0, keepdims=True)
    g2 = jnp.dot(x_ref[...], ct2, preferred_element_type=jnp.float32,
                 precision=jax.lax.Precision.HIGHEST)
    score = cnorm + g2
    m = jnp.min(score, axis=1, keepdims=True)
    col = jax.lax.broadcasted_iota(jnp.int32, score.shape, 1)
    idx = jnp.min(jnp.where(score == m, col, k), axis=1)
    out_ref[...] = idx


def kernel(x, centroids):
    n, d = x.shape
    return pl.pallas_call(
        _vq_argmin_kernel,
        out_shape=jax.ShapeDtypeStruct((n,), jnp.int32),
    )(x, centroids)
